# baseline (speedup 1.0000x reference)
import functools

import jax
import jax.numpy as jnp
import numpy as np
from jax import lax
from jax.experimental import pallas as pl
from jax.experimental.pallas import tpu as pltpu
from jax.experimental.pallas import tpu_sc as plsc

_B = 128
_N = 512
_L = 16
_NC = 2
_NS = 16
_NW = _NC * _NS
_RPW = _B // _NW
_CHUNKS = _N // _L

_mesh = plsc.VectorSubcoreMesh(
    core_axis_name="c", subcore_axis_name="s", num_cores=_NC, num_subcores=_NS
)


@functools.partial(
    pl.kernel,
    out_type=jax.ShapeDtypeStruct((_B, 2, _N), jnp.float32),
    mesh=_mesh,
    scratch_types=[
        pltpu.VMEM((_RPW, _N), jnp.float32),
        pltpu.VMEM((_RPW, 2, _N), jnp.float32),
        pltpu.SemaphoreType.DMA,
        pltpu.SemaphoreType.DMA,
        pltpu.SemaphoreType.DMA,
    ],
    compiler_params=pltpu.CompilerParams(needs_layout_passes=False),
)
def _compact(x_hbm, idx_hbm, out_hbm, rows_v, out_v, sem1, sem2, sem3):
    wid = lax.axis_index("s") * _NC + lax.axis_index("c")
    base = wid * _RPW
    cp_rows = pltpu.make_async_copy(x_hbm.at[pl.ds(base, _RPW)], rows_v, sem1)
    cp_vals = pltpu.make_async_copy(
        x_hbm.at[pl.ds(base, _RPW)], out_v.at[:, 1, :], sem2
    )
    cp_idx = pltpu.make_async_copy(idx_hbm, out_v.at[:, 0, :], sem3)
    cp_rows.start()
    cp_vals.start()
    cp_idx.start()
    cp_rows.wait()

    zf = jnp.zeros((_L,), jnp.float32)

    def row_body(r, _):
        def det_body(c, anyz):
            return anyz | (rows_v[r, pl.ds(c * _L, _L)] == 0.0)

        anyz = lax.fori_loop(
            0, _CHUNKS, det_body, jnp.zeros((_L,), jnp.bool_), unroll=8
        )
        haszero = jnp.any(anyz)

        @pl.when(haszero)
        def _general():
            r_splat = jnp.full((_L,), r, jnp.int32)
            plane0 = jnp.zeros((_L,), jnp.int32)
            plane1 = jnp.ones((_L,), jnp.int32)
            iota_f = lax.iota(jnp.int32, _L).astype(jnp.float32)

            def zero_body(c, _):
                out_v[r, 0, pl.ds(c * _L, _L)] = zf
                out_v[r, 1, pl.ds(c * _L, _L)] = zf
                return 0

            lax.fori_loop(0, _CHUNKS, zero_body, 0, unroll=4)

            def pack_body(c, n_off):
                sl = pl.ds(c * _L, _L)
                v = rows_v[r, sl]
                m = v != 0.0
                dest = plsc.cumsum(m.astype(jnp.int32)) + n_off
                idx_f = iota_f + (c * _L).astype(jnp.float32)
                plsc.store_scatter(
                    out_v, [r_splat, plane0, dest], idx_f, mask=m
                )
                plsc.store_scatter(out_v, [r_splat, plane1, dest], v, mask=m)
                return n_off + plsc.all_reduce_population_count(m)

            lax.fori_loop(
                0, _CHUNKS, pack_body, jnp.full((_L,), -1, jnp.int32)
            )

        return 0

    cp_vals.wait()
    cp_idx.wait()
    lax.fori_loop(0, _RPW, row_body, 0)
    pltpu.sync_copy(out_v, out_hbm.at[pl.ds(base, _RPW)])


_IDX_CONST = np.ascontiguousarray(
    np.broadcast_to(np.arange(_N, dtype=np.float32), (_RPW, _N))
)


def kernel(mlm_logits):
    return _compact(mlm_logits, jnp.asarray(_IDX_CONST))
